# baseline (speedup 1.0000x reference)
import jax
import jax.numpy as jnp
from jax import lax
from jax.experimental import pallas as pl
from jax.experimental.pallas import tpu as pltpu
from jax.experimental.pallas import tpu_sc as plsc

B = 16384
D = 512
SEG = 64
NSEG = 8
V = 100000

NC = 2
NS = 16
NW = NC * NS
L = 16

B_PER_W = B // NW
CHUNK = 64
HALF = 16
NH = CHUNK // HALF
N_CHUNKS = B_PER_W // CHUNK
N_PAIRS = N_CHUNKS // 2

_GATHER_DNUMS = lax.GatherDimensionNumbers(
    offset_dims=(), collapsed_slice_dims=(0,), start_index_map=(0,))


def _rot_add(v, rots):
    for r in rots:
        v = v + lax.gather(v, r[:, None], _GATHER_DNUMS, slice_sizes=(1,),
                           mode=lax.GatherScatterMode.PROMISE_IN_BOUNDS)
    return v


def _rsqrt(v):
    i = plsc.bitcast(v, jnp.int32)
    i = jnp.int32(0x5F3759DF) - lax.shift_right_logical(i, 1)
    y = plsc.bitcast(i, jnp.float32)
    half = v * jnp.float32(0.5)
    for _ in range(2):
        y = y * (jnp.float32(1.5) - half * y * y)
    return y


def _body(idx_hbm, table_hbm, gamma_hbm, beta_hbm, out_hbm,
          idx_v, buf0, buf1, obuf0, obuf1, gamma_v, beta_v,
          in0, in1, ob0, ob1):
    wid = lax.axis_index("c") * NS + lax.axis_index("s")
    base = wid * B_PER_W

    pltpu.sync_copy(idx_hbm.at[pl.ds(base, B_PER_W)], idx_v)
    pltpu.sync_copy(gamma_hbm, gamma_v)
    pltpu.sync_copy(beta_hbm, beta_v)

    obufs = (obuf0, obuf1)
    osems = (ob0, ob1)

    iota = lax.iota(jnp.int32, L)
    rots = [(iota + k) % L for k in (8, 4, 2, 1)]
    gv = [gamma_v[pl.ds(16 * k, 16)] for k in range(4)]
    bv = [beta_v[pl.ds(16 * k, 16)] for k in range(4)]
    inv_seg = jnp.float32(1.0 / SEG)
    eps = jnp.float32(1e-5)

    def start_gather(c, buf, sem):
        pltpu.make_async_copy(
            table_hbm.at[idx_v.at[pl.ds(c * CHUNK, CHUNK)]], buf, sem
        ).start()

    def wait_gather(c, buf, sem):
        pltpu.make_async_copy(
            table_hbm.at[idx_v.at[pl.ds(c * CHUNK, CHUNK)]], buf, sem
        ).wait()

    def start_out(c, h, obuf, sem):
        pltpu.make_async_copy(
            obuf, out_hbm.at[pl.ds(base + c * CHUNK + h * HALF, HALF)], sem
        ).start()

    def wait_out(c, h, obuf, sem):
        pltpu.make_async_copy(
            obuf, out_hbm.at[pl.ds(base + c * CHUNK + h * HALF, HALF)], sem
        ).wait()

    def layernorm_chunk(c, buf):
        for h in range(NH):
            obuf = obufs[h % 2]
            sem = osems[h % 2]

            if h >= 2:
                wait_out(c, h, obuf, sem)
            else:
                @pl.when(c > 0)
                def _():
                    wait_out(c, h, obuf, sem)

            def row_step(r, _):
                rr = h * HALF + r
                for s in range(NSEG):
                    xs = [buf[rr, pl.ds(s * SEG + 16 * k, 16)]
                          for k in range(4)]
                    ssum = _rot_add(xs[0] + xs[1] + xs[2] + xs[3], rots)
                    qsum = _rot_add(xs[0] * xs[0] + xs[1] * xs[1]
                                    + xs[2] * xs[2] + xs[3] * xs[3], rots)
                    mean = ssum * inv_seg
                    var = qsum * inv_seg - mean * mean
                    rstd = _rsqrt(var + eps)
                    for k in range(4):
                        obuf[r, s, pl.ds(16 * k, 16)] = (
                            (xs[k] - mean) * rstd * gv[k] + bv[k])
                return 0

            lax.fori_loop(0, HALF, row_step, 0)
            start_out(c, h, obuf, sem)

    start_gather(0, buf0, in0)
    start_gather(1, buf1, in1)

    def pair_step(t, _):
        c0 = 2 * t
        c1 = c0 + 1

        wait_gather(c0, buf0, in0)
        layernorm_chunk(c0, buf0)

        @pl.when(t < N_PAIRS - 1)
        def _():
            start_gather(c0 + 2, buf0, in0)

        wait_gather(c1, buf1, in1)
        layernorm_chunk(c1, buf1)

        @pl.when(t < N_PAIRS - 1)
        def _():
            start_gather(c1 + 2, buf1, in1)

        return 0

    lax.fori_loop(0, N_PAIRS, pair_step, 0)
    wait_out(N_CHUNKS - 1, NH - 2, obuf0, ob0)
    wait_out(N_CHUNKS - 1, NH - 1, obuf1, ob1)


@jax.jit
def _encode(prompt_idx, table, gamma, beta):
    mesh = plsc.VectorSubcoreMesh(core_axis_name="c", subcore_axis_name="s")
    run = pl.kernel(
        _body,
        out_type=jax.ShapeDtypeStruct((B, NSEG, SEG), jnp.float32),
        mesh=mesh,
        compiler_params=pltpu.CompilerParams(needs_layout_passes=False),
        scratch_types=[
            pltpu.VMEM((B_PER_W,), jnp.int32),
            pltpu.VMEM((CHUNK, D), jnp.float32),
            pltpu.VMEM((CHUNK, D), jnp.float32),
            pltpu.VMEM((HALF, NSEG, SEG), jnp.float32),
            pltpu.VMEM((HALF, NSEG, SEG), jnp.float32),
            pltpu.VMEM((SEG,), jnp.float32),
            pltpu.VMEM((SEG,), jnp.float32),
            pltpu.SemaphoreType.DMA,
            pltpu.SemaphoreType.DMA,
            pltpu.SemaphoreType.DMA,
            pltpu.SemaphoreType.DMA,
        ],
    )
    return run(prompt_idx, table, gamma, beta)


def kernel(prompt_idx, table, gamma, beta):
    return _encode(prompt_idx, table, gamma, beta)


# speedup vs baseline: 2.0094x; 2.0094x over previous
import jax
import jax.numpy as jnp
from jax import lax
from jax.experimental import pallas as pl
from jax.experimental.pallas import tpu as pltpu
from jax.experimental.pallas import tpu_sc as plsc

B = 16384
D = 512
SEG = 64
NSEG = 8
V = 100000

NC = 2
NS = 16
NW = NC * NS
L = 16

B_PER_W = B // NW
CHUNK = 64
N_CHUNKS = B_PER_W // CHUNK
N_PAIRS = N_CHUNKS // 2


def _rsqrt(v):
    i = plsc.bitcast(v, jnp.int32)
    i = jnp.int32(0x5F3759DF) - lax.shift_right_logical(i, 1)
    y = plsc.bitcast(i, jnp.float32)
    half = v * jnp.float32(0.5)
    for _ in range(2):
        y = y * (jnp.float32(1.5) - half * y * y)
    return y


def _body(idx_hbm, table_hbm, gamma_hbm, beta_hbm, out_hbm,
          idx_v, buf0, buf1, gamma_v, beta_v,
          in0, in1, out0, out1):
    wid = lax.axis_index("c") * NS + lax.axis_index("s")
    base = wid * B_PER_W

    pltpu.sync_copy(idx_hbm.at[pl.ds(base, B_PER_W)], idx_v)
    pltpu.sync_copy(gamma_hbm, gamma_v)
    pltpu.sync_copy(beta_hbm, beta_v)

    gv = [gamma_v[pl.ds(16 * k, 16)] for k in range(4)]
    bv = [beta_v[pl.ds(16 * k, 16)] for k in range(4)]
    inv_seg = jnp.float32(1.0 / SEG)
    eps = jnp.float32(1e-5)

    def start_gather(c, buf, sem):
        pltpu.make_async_copy(
            table_hbm.at[idx_v.at[pl.ds(c * CHUNK, CHUNK)]], buf, sem
        ).start()

    def wait_gather(c, buf, sem):
        pltpu.make_async_copy(
            table_hbm.at[idx_v.at[pl.ds(c * CHUNK, CHUNK)]], buf, sem
        ).wait()

    def start_out(c, buf, sem):
        pltpu.make_async_copy(
            buf, out_hbm.at[pl.ds(base + c * CHUNK, CHUNK)], sem
        ).start()

    def wait_out(c, buf, sem):
        pltpu.make_async_copy(
            buf, out_hbm.at[pl.ds(base + c * CHUNK, CHUNK)], sem
        ).wait()

    def layernorm_chunk(buf):
        def row_step(r, _):
            for s in range(NSEG):
                xs = [buf[r, pl.ds(s * SEG + 16 * k, 16)] for k in range(4)]
                ssum = jnp.sum(xs[0] + xs[1] + xs[2] + xs[3])
                qsum = jnp.sum(xs[0] * xs[0] + xs[1] * xs[1]
                               + xs[2] * xs[2] + xs[3] * xs[3])
                mean = jnp.full((L,), ssum * inv_seg)
                var = jnp.full((L,), qsum * inv_seg) - mean * mean
                rstd = _rsqrt(var + eps)
                for k in range(4):
                    buf[r, pl.ds(s * SEG + 16 * k, 16)] = (
                        (xs[k] - mean) * rstd * gv[k] + bv[k])
            return 0

        lax.fori_loop(0, CHUNK, row_step, 0)

    start_gather(0, buf0, in0)

    def pair_step(t, _):
        c0 = 2 * t
        c1 = c0 + 1

        @pl.when(t > 0)
        def _():
            wait_out(c0 - 1, buf1, out1)

        start_gather(c1, buf1, in1)
        wait_gather(c0, buf0, in0)
        layernorm_chunk(buf0)
        start_out(c0, buf0, out0)

        wait_gather(c1, buf1, in1)
        layernorm_chunk(buf1)
        start_out(c1, buf1, out1)

        @pl.when(t < N_PAIRS - 1)
        def _():
            wait_out(c0, buf0, out0)
            start_gather(c0 + 2, buf0, in0)

        return 0

    lax.fori_loop(0, N_PAIRS, pair_step, 0)
    wait_out(N_CHUNKS - 2, buf0, out0)
    wait_out(N_CHUNKS - 1, buf1, out1)


@jax.jit
def _encode(prompt_idx, table, gamma, beta):
    mesh = plsc.VectorSubcoreMesh(core_axis_name="c", subcore_axis_name="s")
    run = pl.kernel(
        _body,
        out_type=jax.ShapeDtypeStruct((B, D), jnp.float32),
        mesh=mesh,
        compiler_params=pltpu.CompilerParams(needs_layout_passes=False),
        scratch_types=[
            pltpu.VMEM((B_PER_W,), jnp.int32),
            pltpu.VMEM((CHUNK, D), jnp.float32),
            pltpu.VMEM((CHUNK, D), jnp.float32),
            pltpu.VMEM((SEG,), jnp.float32),
            pltpu.VMEM((SEG,), jnp.float32),
            pltpu.SemaphoreType.DMA,
            pltpu.SemaphoreType.DMA,
            pltpu.SemaphoreType.DMA,
            pltpu.SemaphoreType.DMA,
        ],
    )
    return run(prompt_idx, table, gamma, beta)


def kernel(prompt_idx, table, gamma, beta):
    out = _encode(prompt_idx, table, gamma, beta)
    return out.reshape(B, NSEG, SEG)


# speedup vs baseline: 2.0510x; 1.0207x over previous
import jax
import jax.numpy as jnp
from jax import lax
from jax.experimental import pallas as pl
from jax.experimental.pallas import tpu as pltpu
from jax.experimental.pallas import tpu_sc as plsc

B = 16384
D = 512
SEG = 64
NSEG = 8
V = 100000

NC = 2
NS = 16
NW = NC * NS
L = 16

B_PER_W = B // NW
CHUNK = 64
N_CHUNKS = B_PER_W // CHUNK
N_PAIRS = N_CHUNKS // 2
NEWTON_ITERS = 2


def _rsqrt(v):
    i = plsc.bitcast(v, jnp.int32)
    i = jnp.int32(0x5F3759DF) - lax.shift_right_logical(i, 1)
    y = plsc.bitcast(i, jnp.float32)
    half = v * jnp.float32(0.5)
    for _ in range(NEWTON_ITERS):
        y = y * (jnp.float32(1.5) - half * y * y)
    return y


def _body(idx_hbm, table_hbm, gamma_hbm, beta_hbm, out_hbm,
          idx_v, buf0, buf1, gamma_v, beta_v,
          in0, in1, out0, out1):
    wid = lax.axis_index("c") * NS + lax.axis_index("s")
    base = wid * B_PER_W

    pltpu.sync_copy(idx_hbm.at[pl.ds(base, B_PER_W)], idx_v)
    pltpu.sync_copy(gamma_hbm, gamma_v)
    pltpu.sync_copy(beta_hbm, beta_v)

    gv = [gamma_v[pl.ds(16 * k, 16)] for k in range(4)]
    bv = [beta_v[pl.ds(16 * k, 16)] for k in range(4)]
    inv_seg = jnp.float32(1.0 / SEG)
    eps = jnp.float32(1e-5)

    def start_gather(c, buf, sem):
        pltpu.make_async_copy(
            table_hbm.at[idx_v.at[pl.ds(c * CHUNK, CHUNK)]], buf, sem
        ).start()

    def wait_gather(c, buf, sem):
        pltpu.make_async_copy(
            table_hbm.at[idx_v.at[pl.ds(c * CHUNK, CHUNK)]], buf, sem
        ).wait()

    def start_out(c, buf, sem):
        pltpu.make_async_copy(
            buf, out_hbm.at[pl.ds(base + c * CHUNK, CHUNK)], sem
        ).start()

    def wait_out(c, buf, sem):
        pltpu.make_async_copy(
            buf, out_hbm.at[pl.ds(base + c * CHUNK, CHUNK)], sem
        ).wait()

    def layernorm_chunk(buf):
        def row_step(r, _):
            for s in range(NSEG):
                xs = [buf[r, pl.ds(s * SEG + 16 * k, 16)] for k in range(4)]
                ssum = jnp.sum(xs[0] + xs[1] + xs[2] + xs[3])
                qsum = jnp.sum(xs[0] * xs[0] + xs[1] * xs[1]
                               + xs[2] * xs[2] + xs[3] * xs[3])
                mean_s = ssum * inv_seg
                var_s = qsum * inv_seg - mean_s * mean_s
                mean = jnp.full((L,), mean_s)
                rstd = _rsqrt(jnp.full((L,), var_s + eps))
                for k in range(4):
                    buf[r, pl.ds(s * SEG + 16 * k, 16)] = (
                        (xs[k] - mean) * rstd * gv[k] + bv[k])
            return 0

        lax.fori_loop(0, CHUNK, row_step, 0)

    start_gather(0, buf0, in0)

    def pair_step(t, _):
        c0 = 2 * t
        c1 = c0 + 1

        @pl.when(t > 0)
        def _():
            wait_out(c0 - 1, buf1, out1)

        start_gather(c1, buf1, in1)
        wait_gather(c0, buf0, in0)
        layernorm_chunk(buf0)
        start_out(c0, buf0, out0)

        wait_gather(c1, buf1, in1)
        layernorm_chunk(buf1)
        start_out(c1, buf1, out1)

        @pl.when(t < N_PAIRS - 1)
        def _():
            wait_out(c0, buf0, out0)
            start_gather(c0 + 2, buf0, in0)

        return 0

    lax.fori_loop(0, N_PAIRS, pair_step, 0)
    wait_out(N_CHUNKS - 2, buf0, out0)
    wait_out(N_CHUNKS - 1, buf1, out1)


@jax.jit
def _encode(prompt_idx, table, gamma, beta):
    mesh = plsc.VectorSubcoreMesh(core_axis_name="c", subcore_axis_name="s")
    run = pl.kernel(
        _body,
        out_type=jax.ShapeDtypeStruct((B, D), jnp.float32),
        mesh=mesh,
        compiler_params=pltpu.CompilerParams(needs_layout_passes=False),
        scratch_types=[
            pltpu.VMEM((B_PER_W,), jnp.int32),
            pltpu.VMEM((CHUNK, D), jnp.float32),
            pltpu.VMEM((CHUNK, D), jnp.float32),
            pltpu.VMEM((SEG,), jnp.float32),
            pltpu.VMEM((SEG,), jnp.float32),
            pltpu.SemaphoreType.DMA,
            pltpu.SemaphoreType.DMA,
            pltpu.SemaphoreType.DMA,
            pltpu.SemaphoreType.DMA,
        ],
    )
    return run(prompt_idx, table, gamma, beta)


def kernel(prompt_idx, table, gamma, beta):
    out = _encode(prompt_idx, table, gamma, beta)
    return out.reshape(B, NSEG, SEG)


# speedup vs baseline: 2.1558x; 1.0511x over previous
import jax
import jax.numpy as jnp
from jax import lax
from jax.experimental import pallas as pl
from jax.experimental.pallas import tpu as pltpu
from jax.experimental.pallas import tpu_sc as plsc

B = 16384
D = 512
SEG = 64
NSEG = 8
V = 100000

NC = 2
NS = 16
NW = NC * NS
L = 16

B_PER_W = B // NW
CHUNK = 64
N_CHUNKS = B_PER_W // CHUNK
N_PAIRS = N_CHUNKS // 2
NEWTON_ITERS = 1


def _rsqrt(v):
    i = plsc.bitcast(v, jnp.int32)
    i = jnp.int32(0x5F3759DF) - lax.shift_right_logical(i, 1)
    y = plsc.bitcast(i, jnp.float32)
    half = v * jnp.float32(0.5)
    for _ in range(NEWTON_ITERS):
        y = y * (jnp.float32(1.5) - half * y * y)
    return y


def _body(idx_hbm, table_hbm, gamma_hbm, beta_hbm, out_hbm,
          idx_v, buf0, buf1, gamma_v, beta_v,
          in0, in1, out0, out1):
    wid = lax.axis_index("c") * NS + lax.axis_index("s")
    base = wid * B_PER_W

    pltpu.sync_copy(idx_hbm.at[pl.ds(base, B_PER_W)], idx_v)
    pltpu.sync_copy(gamma_hbm, gamma_v)
    pltpu.sync_copy(beta_hbm, beta_v)

    gv = [gamma_v[pl.ds(16 * k, 16)] for k in range(4)]
    bv = [beta_v[pl.ds(16 * k, 16)] for k in range(4)]
    inv_seg = jnp.float32(1.0 / SEG)
    eps = jnp.float32(1e-5)

    def start_gather(c, buf, sem):
        pltpu.make_async_copy(
            table_hbm.at[idx_v.at[pl.ds(c * CHUNK, CHUNK)]], buf, sem
        ).start()

    def wait_gather(c, buf, sem):
        pltpu.make_async_copy(
            table_hbm.at[idx_v.at[pl.ds(c * CHUNK, CHUNK)]], buf, sem
        ).wait()

    def start_out(c, buf, sem):
        pltpu.make_async_copy(
            buf, out_hbm.at[pl.ds(base + c * CHUNK, CHUNK)], sem
        ).start()

    def wait_out(c, buf, sem):
        pltpu.make_async_copy(
            buf, out_hbm.at[pl.ds(base + c * CHUNK, CHUNK)], sem
        ).wait()

    def layernorm_chunk(buf):
        def row_step(r, _):
            for s in range(NSEG):
                xs = [buf[r, pl.ds(s * SEG + 16 * k, 16)] for k in range(4)]
                ssum = jnp.sum(xs[0] + xs[1] + xs[2] + xs[3])
                qsum = jnp.sum(xs[0] * xs[0] + xs[1] * xs[1]
                               + xs[2] * xs[2] + xs[3] * xs[3])
                mean_s = ssum * inv_seg
                var_s = qsum * inv_seg - mean_s * mean_s
                mean = jnp.full((L,), mean_s)
                rstd = _rsqrt(jnp.full((L,), var_s + eps))
                for k in range(4):
                    buf[r, pl.ds(s * SEG + 16 * k, 16)] = (
                        (xs[k] - mean) * rstd * gv[k] + bv[k])
            return 0

        lax.fori_loop(0, CHUNK, row_step, 0)

    start_gather(0, buf0, in0)

    def pair_step(t, _):
        c0 = 2 * t
        c1 = c0 + 1

        @pl.when(t > 0)
        def _():
            wait_out(c0 - 1, buf1, out1)

        start_gather(c1, buf1, in1)
        wait_gather(c0, buf0, in0)
        layernorm_chunk(buf0)
        start_out(c0, buf0, out0)

        wait_gather(c1, buf1, in1)
        layernorm_chunk(buf1)
        start_out(c1, buf1, out1)

        @pl.when(t < N_PAIRS - 1)
        def _():
            wait_out(c0, buf0, out0)
            start_gather(c0 + 2, buf0, in0)

        return 0

    lax.fori_loop(0, N_PAIRS, pair_step, 0)
    wait_out(N_CHUNKS - 2, buf0, out0)
    wait_out(N_CHUNKS - 1, buf1, out1)


@jax.jit
def _encode(prompt_idx, table, gamma, beta):
    mesh = plsc.VectorSubcoreMesh(core_axis_name="c", subcore_axis_name="s")
    run = pl.kernel(
        _body,
        out_type=jax.ShapeDtypeStruct((B, D), jnp.float32),
        mesh=mesh,
        compiler_params=pltpu.CompilerParams(needs_layout_passes=False),
        scratch_types=[
            pltpu.VMEM((B_PER_W,), jnp.int32),
            pltpu.VMEM((CHUNK, D), jnp.float32),
            pltpu.VMEM((CHUNK, D), jnp.float32),
            pltpu.VMEM((SEG,), jnp.float32),
            pltpu.VMEM((SEG,), jnp.float32),
            pltpu.SemaphoreType.DMA,
            pltpu.SemaphoreType.DMA,
            pltpu.SemaphoreType.DMA,
            pltpu.SemaphoreType.DMA,
        ],
    )
    return run(prompt_idx, table, gamma, beta)


def kernel(prompt_idx, table, gamma, beta):
    out = _encode(prompt_idx, table, gamma, beta)
    return out.reshape(B, NSEG, SEG)
